# Initial kernel scaffold; baseline (speedup 1.0000x reference)
import jax
import jax.numpy as jnp
from jax.experimental import pallas as pl


def kernel(variantxgene_embedding, variantxgene_ixs, W, b, variantxgene_effect):
    raise NotImplementedError("write your pallas kernel here")


# speedup vs baseline: 15.5589x; 15.5589x over previous
import functools

import jax
import jax.numpy as jnp
from jax import lax
from jax.experimental import pallas as pl
from jax.experimental.pallas import tpu as pltpu, tpu_sc as plsc

N_EMB = 16
N_VXG = 1000000
B = 16384
L = 200
N = B * L

NC, NS = 2, 16
NW = NC * NS
PER_W = N // NW
CHUNK = 2048
CHUNKS = PER_W // CHUNK
GROUPS = CHUNK // 16
IDX_ROWS = CHUNK // 128

_mesh = plsc.VectorSubcoreMesh(core_axis_name="c", subcore_axis_name="s")


def _body(emb_hbm, ixs_hbm, wb_hbm, eff_hbm, out_hbm,
          emb_v, idx_v, val_v, out_v, wb_v, gsem):
    wid = lax.axis_index("c") * NS + lax.axis_index("s")

    pltpu.sync_copy(wb_hbm, wb_v)

    def chunk_body(g, carry):
        row_base = pl.multiple_of(wid * PER_W + g * CHUNK, CHUNK)
        pltpu.sync_copy(emb_hbm.at[pl.ds(pl.multiple_of(row_base * N_EMB, 8),
                                         CHUNK * N_EMB)],
                        emb_v)
        pltpu.sync_copy(ixs_hbm.at[pl.ds(pl.multiple_of(row_base // 128, 8),
                                         IDX_ROWS)], idx_v)
        cps = [pltpu.async_copy(eff_hbm.at[idx_v.at[j]],
                                val_v.at[pl.ds(j * 128, 128)], gsem)
               for j in range(IDX_ROWS)]
        for cp in cps:
            cp.wait()

        def grp(i, c):
            base = i * (16 * N_EMB)
            lane16 = lax.iota(jnp.int32, 16) * 16
            bias = plsc.load_gather(wb_v, [jnp.full((16,), N_EMB, jnp.int32)])

            def kbody(kk, a):
                k4 = kk * 4
                for dk in range(4):
                    vals = plsc.load_gather(emb_v,
                                            [lane16 + (base + k4 + dk)])
                    wk = plsc.load_gather(
                        wb_v, [jnp.zeros((16,), jnp.int32) + (k4 + dk)])
                    a = a + vals * wk
                return a

            acc = lax.fori_loop(0, N_EMB // 4, kbody, bias)
            out_v[pl.ds(i * 16, 16)] = val_v[pl.ds(i * 16, 16)] * jnp.exp(acc)
            return c

        lax.fori_loop(0, GROUPS, grp, 0)
        pltpu.sync_copy(out_v, out_hbm.at[pl.ds(row_base, CHUNK)])
        return carry

    lax.fori_loop(0, CHUNKS, chunk_body, 0)


_sc_call = pl.kernel(
    _body,
    out_type=jax.ShapeDtypeStruct((N,), jnp.float32),
    mesh=_mesh,
    compiler_params=pltpu.CompilerParams(needs_layout_passes=False),
    scratch_types=[
        pltpu.VMEM((CHUNK * N_EMB,), jnp.float32),
        pltpu.VMEM((IDX_ROWS, 128), jnp.int32),
        pltpu.VMEM((CHUNK,), jnp.float32),
        pltpu.VMEM((CHUNK,), jnp.float32),
        pltpu.VMEM((128,), jnp.float32),
        pltpu.SemaphoreType.DMA,
    ],
)


def kernel(variantxgene_embedding, variantxgene_ixs, W, b, variantxgene_effect):
    emb_flat = variantxgene_embedding.reshape(N * N_EMB)
    ixs2 = variantxgene_ixs.reshape(N // 128, 128).astype(jnp.int32)
    wb = jnp.concatenate([W.reshape(N_EMB), b.reshape(1),
                          jnp.zeros(111, jnp.float32)])
    out_flat = _sc_call(emb_flat, ixs2, wb, variantxgene_effect)
    return out_flat.reshape(B, L)
